# baseline (speedup 1.0000x reference)
import functools

import numpy as np
import jax
import jax.numpy as jnp
from jax import lax
from jax.experimental import pallas as pl
from jax.experimental.pallas import tpu as pltpu
from jax.experimental.pallas import tpu_sc as plsc

N = 10000
E = 320000
IN_C = 128
HID = 256
OUT_C = 64
L = 4
ALPHA = 0.1
THETA = 0.5
EPS = 1e-5

NC = 2
NS = 16
FH = HID // 2
CH = 128
PAD = 16
A_OFF = 624
A_SZ = 640

EPT_PROP = E // NS
GROUP = 16
KCH_PROP = -(-EPT_PROP // (CH * GROUP)) * GROUP
NG = KCH_PROP // GROUP
EPT_DEG = E // (NC * NS)
KCH_DEG = -(-EPT_DEG // CH)

_MESH = plsc.VectorSubcoreMesh(core_axis_name="c", subcore_axis_name="s")
_SC_PARAMS = pltpu.CompilerParams(use_tc_tiling_on_sc=False)


def _sc_degree_body(colpad, zeros16, e0, degacc_out, col_v, e0_v, acc_sh, sem):
    c = lax.axis_index("c")
    s = lax.axis_index("s")
    w = c * NS + s
    pltpu.sync_copy(zeros16.at[pl.ds(s * A_OFF, A_SZ)],
                    acc_sh.at[pl.ds(s * A_OFF, A_SZ)])
    pltpu.sync_copy(e0, e0_v)
    pltpu.sync_copy(colpad.at[w], col_v)
    plsc.subcore_barrier()

    def body(k, carry):
        pltpu.sync_copy(e0_v, acc_sh.at[col_v.at[k]], add=True)
        return carry

    lax.fori_loop(0, KCH_DEG, body, 0)
    plsc.subcore_barrier()
    pltpu.sync_copy(acc_sh.at[pl.ds(s * A_OFF, A_SZ)],
                    degacc_out.at[c, pl.ds(s * A_OFF, A_SZ)])


_sc_degree = pl.kernel(
    _sc_degree_body,
    out_type=jax.ShapeDtypeStruct((NC, N, 16), jnp.float32),
    mesh=_MESH,
    scratch_types=[
        pltpu.VMEM((KCH_DEG, CH), jnp.int32),
        pltpu.VMEM((CH, 16), jnp.float32),
        pltpu.VMEM_SHARED((N + PAD, 16), jnp.float32),
        pltpu.SemaphoreType.DMA,
    ],
    compiler_params=_SC_PARAMS,
)


def _sc_prop_body(hs, idx2g, seg_out, ig0, ig1, r0, r1, acc_sh, sg, ss, si):
    c = lax.axis_index("c")
    s = lax.axis_index("s")
    base = (c * NS + s) * NG

    def wait_g():
        pltpu.make_async_copy(hs.at[pl.ds(0, CH)], r0, sg).wait()

    def wait_s():
        pltpu.make_async_copy(hs.at[pl.ds(0, CH)], r0, ss).wait()

    def wait_i():
        pltpu.make_async_copy(idx2g.at[0], ig0, si).wait()

    def do_group(h, cur, nxt, first, last):
        for j in range(GROUP):
            rp, rq = (r0, r1) if j % 2 == 0 else (r1, r0)
            wait_g()
            if not (first and j == 0):
                wait_s()
            if j == 0 and not (first or last):
                pltpu.async_copy(idx2g.at[base + h + 1], nxt, si)
            pltpu.async_copy(rp, acc_sh.at[cur.at[2 * j + 1]], ss, add=True)
            if j < GROUP - 1:
                pltpu.async_copy(hs.at[cur.at[2 * j + 2]], rq, sg)
            elif not last:
                wait_i()
                pltpu.async_copy(hs.at[nxt.at[0]], rq, sg)

    pltpu.sync_copy(hs.at[pl.ds(c * N + s * A_OFF, A_SZ)],
                    acc_sh.at[pl.ds(s * A_OFF, A_SZ)])
    plsc.subcore_barrier()

    pltpu.sync_copy(idx2g.at[base], ig0)
    pltpu.async_copy(idx2g.at[base + 1], ig1, si)
    pltpu.async_copy(hs.at[ig0.at[0]], r0, sg)
    do_group(0, ig0, ig1, True, False)

    def pair(i2, carry):
        h1 = 1 + 2 * i2
        do_group(h1, ig1, ig0, False, False)
        do_group(h1 + 1, ig0, ig1, False, False)
        return carry

    lax.fori_loop(0, (NG - 2) // 2, pair, 0)
    do_group(NG - 1, ig1, ig0, False, True)
    wait_s()

    plsc.subcore_barrier()
    pltpu.sync_copy(acc_sh.at[pl.ds(s * A_OFF, A_SZ)],
                    seg_out.at[pl.ds(c * N + s * A_OFF, A_SZ)])


_sc_prop = pl.kernel(
    _sc_prop_body,
    out_type=jax.ShapeDtypeStruct((2 * N, FH), jnp.float32),
    mesh=_MESH,
    scratch_types=[
        pltpu.VMEM((GROUP * 2, CH), jnp.int32),
        pltpu.VMEM((GROUP * 2, CH), jnp.int32),
        pltpu.VMEM((CH, FH), jnp.float32),
        pltpu.VMEM((CH, FH), jnp.float32),
        pltpu.VMEM_SHARED((N + PAD, FH), jnp.float32),
        pltpu.SemaphoreType.DMA,
        pltpu.SemaphoreType.DMA,
        pltpu.SemaphoreType.DMA,
    ],
    compiler_params=_SC_PARAMS,
)


BM = 2000


def _tc1_body(x_ref, w0_ref, b0_ref, da_ref, h0_ref, hs_ref, dis_ref):
    da = da_ref[...]
    deg = da[0][:, 0:1] + da[1][:, 0:1] + 1.0
    dis = lax.rsqrt(deg)
    h = jnp.dot(x_ref[...], w0_ref[...], preferred_element_type=jnp.float32)
    h = jnp.maximum(h + b0_ref[...], 0.0)
    h0_ref[...] = h
    dis_ref[...] = dis
    hs_ref[0] = h[:, :FH] * dis
    hs_ref[1] = h[:, FH:] * dis


def _tc1(x, W0, b0, degacc):
    return pl.pallas_call(
        _tc1_body,
        grid=(N // BM,),
        in_specs=[
            pl.BlockSpec((BM, IN_C), lambda i: (i, 0)),
            pl.BlockSpec((IN_C, HID), lambda i: (0, 0)),
            pl.BlockSpec((1, HID), lambda i: (0, 0)),
            pl.BlockSpec((NC, BM, 16), lambda i: (0, i, 0)),
        ],
        out_specs=[
            pl.BlockSpec((BM, HID), lambda i: (i, 0)),
            pl.BlockSpec((2, BM, FH), lambda i: (0, i, 0)),
            pl.BlockSpec((BM, 1), lambda i: (i, 0)),
        ],
        out_shape=[
            jax.ShapeDtypeStruct((N, HID), jnp.float32),
            jax.ShapeDtypeStruct((2, N, FH), jnp.float32),
            jax.ShapeDtypeStruct((N, 1), jnp.float32),
        ],
    )(x, W0, b0, degacc)


def _tc_layer_body(bl, seg_ref, h0_ref, dis_ref, w_ref, g_ref, be_ref, rm_ref,
                   rv_ref, hsn_ref):
    dis = dis_ref[...]
    p = jnp.concatenate([seg_ref[0], seg_ref[1]], axis=1) * dis
    t = (1.0 - ALPHA) * p + ALPHA * h0_ref[...]
    u = (1.0 - bl) * t + bl * jnp.dot(t, w_ref[...],
                                      preferred_element_type=jnp.float32)
    scale = g_ref[...] * lax.rsqrt(rv_ref[...] + EPS)
    h = jnp.maximum((u - rm_ref[...]) * scale + be_ref[...], 0.0)
    hsn_ref[0] = h[:, :FH] * dis
    hsn_ref[1] = h[:, FH:] * dis


def _tc_layer(bl, seg, h0, dis, W, g, be, rm, rv):
    return pl.pallas_call(
        functools.partial(_tc_layer_body, bl),
        grid=(N // BM,),
        in_specs=[
            pl.BlockSpec((2, BM, FH), lambda i: (0, i, 0)),
            pl.BlockSpec((BM, HID), lambda i: (i, 0)),
            pl.BlockSpec((BM, 1), lambda i: (i, 0)),
            pl.BlockSpec((HID, HID), lambda i: (0, 0)),
            pl.BlockSpec((1, HID), lambda i: (0, 0)),
            pl.BlockSpec((1, HID), lambda i: (0, 0)),
            pl.BlockSpec((1, HID), lambda i: (0, 0)),
            pl.BlockSpec((1, HID), lambda i: (0, 0)),
        ],
        out_specs=pl.BlockSpec((2, BM, FH), lambda i: (0, i, 0)),
        out_shape=jax.ShapeDtypeStruct((2, N, FH), jnp.float32),
    )(seg, h0, dis, W, g, be, rm, rv)


def _tc_last_body(bl, seg_ref, h0_ref, dis_ref, w_ref, g_ref, be_ref, rm_ref,
                  rv_ref, w1_ref, b1_ref, out_ref):
    dis = dis_ref[...]
    p = jnp.concatenate([seg_ref[0], seg_ref[1]], axis=1) * dis
    t = (1.0 - ALPHA) * p + ALPHA * h0_ref[...]
    u = (1.0 - bl) * t + bl * jnp.dot(t, w_ref[...],
                                      preferred_element_type=jnp.float32)
    scale = g_ref[...] * lax.rsqrt(rv_ref[...] + EPS)
    h = jnp.maximum((u - rm_ref[...]) * scale + be_ref[...], 0.0)
    o = jnp.dot(h, w1_ref[...], preferred_element_type=jnp.float32)
    o = o + b1_ref[...]
    z = o - jnp.max(o, axis=1, keepdims=True)
    out_ref[...] = z - jnp.log(jnp.sum(jnp.exp(z), axis=1, keepdims=True))


def _tc_last(bl, seg, h0, dis, W, g, be, rm, rv, W1, b1):
    return pl.pallas_call(
        functools.partial(_tc_last_body, bl),
        grid=(N // BM,),
        in_specs=[
            pl.BlockSpec((2, BM, FH), lambda i: (0, i, 0)),
            pl.BlockSpec((BM, HID), lambda i: (i, 0)),
            pl.BlockSpec((BM, 1), lambda i: (i, 0)),
            pl.BlockSpec((HID, HID), lambda i: (0, 0)),
            pl.BlockSpec((1, HID), lambda i: (0, 0)),
            pl.BlockSpec((1, HID), lambda i: (0, 0)),
            pl.BlockSpec((1, HID), lambda i: (0, 0)),
            pl.BlockSpec((1, HID), lambda i: (0, 0)),
            pl.BlockSpec((HID, OUT_C), lambda i: (0, 0)),
            pl.BlockSpec((1, OUT_C), lambda i: (0, 0)),
        ],
        out_specs=pl.BlockSpec((BM, OUT_C), lambda i: (i, 0)),
        out_shape=jax.ShapeDtypeStruct((N, OUT_C), jnp.float32),
    )(seg, h0, dis, W, g, be, rm, rv, W1, b1)


def kernel(x, edge_index, W0, b0, Ws, gammas, betas, rmeans, rvars, W1, b1):
    row = edge_index[0]
    col = edge_index[1]

    r16 = jnp.pad(row.reshape(NS, EPT_PROP),
                  ((0, 0), (0, KCH_PROP * CH - EPT_PROP)))
    rowpad = jnp.stack([r16, r16 + N]).reshape(NC, NS, KCH_PROP, CH)
    c16 = jnp.pad(col.reshape(NS, EPT_PROP),
                  ((0, 0), (0, KCH_PROP * CH - EPT_PROP)),
                  constant_values=N).reshape(NS, KCH_PROP, CH)
    colpad = jnp.broadcast_to(c16, (NC, NS, KCH_PROP, CH))
    idx2 = jnp.stack([rowpad, colpad],
                     axis=3).reshape(NC * NS * NG, GROUP * 2, CH)
    colpad_deg = jnp.pad(col.reshape(NC * NS, EPT_DEG),
                         ((0, 0), (0, KCH_DEG * CH - EPT_DEG)),
                         constant_values=N).reshape(NC * NS, KCH_DEG, CH)
    zeros16 = jnp.zeros((N + PAD, 16), jnp.float32)
    e0 = jnp.zeros((CH, 16), jnp.float32).at[:, 0].set(1.0)

    degacc = _sc_degree(colpad_deg, zeros16, e0)
    h0, hs2, dis = _tc1(x, W0, b0.reshape(1, HID), degacc)
    hs = hs2.reshape(2 * N, FH)

    out = None
    for layer in range(L):
        seg = _sc_prop(hs, idx2).reshape(NC, N, FH)
        bl = float(np.log(THETA / (layer + 1) + 1.0))
        g = gammas[layer].reshape(1, HID)
        be = betas[layer].reshape(1, HID)
        rm = rmeans[layer].reshape(1, HID)
        rv = rvars[layer].reshape(1, HID)
        if layer < L - 1:
            hs = _tc_layer(bl, seg, h0, dis, Ws[layer], g, be, rm,
                           rv).reshape(2 * N, FH)
        else:
            out = _tc_last(bl, seg, h0, dis, Ws[layer], g, be, rm, rv, W1,
                           b1.reshape(1, OUT_C))
    return out


# speedup vs baseline: 1.3433x; 1.3433x over previous
import functools

import numpy as np
import jax
import jax.numpy as jnp
from jax import lax
from jax.experimental import pallas as pl
from jax.experimental.pallas import tpu as pltpu
from jax.experimental.pallas import tpu_sc as plsc

N = 10000
E = 320000
IN_C = 128
HID = 256
OUT_C = 64
L = 4
ALPHA = 0.1
THETA = 0.5
EPS = 1e-5

NC = 2
NS = 16
CH = 128
PAD = 16
A_OFF = 624
A_SZ = 640

EPT = E // (NC * NS)
KCH = -(-EPT // CH)
KCH_DEG = KCH

_MESH = plsc.VectorSubcoreMesh(core_axis_name="c", subcore_axis_name="s")
_SC_PARAMS = pltpu.CompilerParams(use_tc_tiling_on_sc=False)


def _sc_degree_body(colpad, zeros16, e0, degacc_out, col_v, e0_v, acc_sh, sem):
    c = lax.axis_index("c")
    s = lax.axis_index("s")
    w = c * NS + s
    pltpu.sync_copy(zeros16.at[pl.ds(s * A_OFF, A_SZ)],
                    acc_sh.at[pl.ds(s * A_OFF, A_SZ)])
    pltpu.sync_copy(e0, e0_v)
    pltpu.sync_copy(colpad.at[w], col_v)
    plsc.subcore_barrier()

    def body(k, carry):
        pltpu.sync_copy(e0_v, acc_sh.at[col_v.at[k]], add=True)
        return carry

    lax.fori_loop(0, KCH_DEG, body, 0)
    plsc.subcore_barrier()
    pltpu.sync_copy(acc_sh.at[pl.ds(s * A_OFF, A_SZ)],
                    degacc_out.at[c, pl.ds(s * A_OFF, A_SZ)])


_sc_degree = pl.kernel(
    _sc_degree_body,
    out_type=jax.ShapeDtypeStruct((NC, N, 16), jnp.float32),
    mesh=_MESH,
    scratch_types=[
        pltpu.VMEM((KCH_DEG, CH), jnp.int32),
        pltpu.VMEM((CH, 16), jnp.float32),
        pltpu.VMEM_SHARED((N + PAD, 16), jnp.float32),
        pltpu.SemaphoreType.DMA,
    ],
    compiler_params=_SC_PARAMS,
)


def _sc_prop_body(hs, z, idx2, seg_out, idx_v, rows_v, acc_sh, sem):
    c = lax.axis_index("c")
    s = lax.axis_index("s")
    w = c * NS + s

    @pl.when(c == 0)
    def _():
        pltpu.sync_copy(hs.at[pl.ds(s * A_OFF, A_SZ)],
                        acc_sh.at[pl.ds(s * A_OFF, A_SZ)])

    @pl.when(c != 0)
    def _():
        pltpu.sync_copy(z.at[pl.ds(s * A_OFF, A_SZ)],
                        acc_sh.at[pl.ds(s * A_OFF, A_SZ)])

    plsc.subcore_barrier()

    def body(k, carry):
        pltpu.sync_copy(idx2.at[w * KCH + k], idx_v)
        pltpu.async_copy(hs.at[idx_v.at[0]], rows_v, sem).wait()
        pltpu.sync_copy(rows_v, acc_sh.at[idx_v.at[1]], add=True)
        return carry

    lax.fori_loop(0, KCH, body, 0)
    plsc.subcore_barrier()
    pltpu.sync_copy(acc_sh.at[pl.ds(s * A_OFF, A_SZ)],
                    seg_out.at[c, pl.ds(s * A_OFF, A_SZ)])


_sc_prop = pl.kernel(
    _sc_prop_body,
    out_type=jax.ShapeDtypeStruct((NC, N, HID), jnp.bfloat16),
    mesh=_MESH,
    scratch_types=[
        pltpu.VMEM((2, CH), jnp.int32),
        pltpu.VMEM((CH, HID), jnp.bfloat16),
        pltpu.VMEM_SHARED((N + PAD, HID), jnp.bfloat16),
        pltpu.SemaphoreType.DMA,
    ],
    compiler_params=_SC_PARAMS,
)


BM = 2000


def _tc1_body(x_ref, w0_ref, b0_ref, da_ref, h0_ref, hs_ref, dis_ref):
    da = da_ref[...]
    deg = da[0][:, 0:1] + da[1][:, 0:1] + 1.0
    dis = lax.rsqrt(deg)
    h = jnp.dot(x_ref[...], w0_ref[...], preferred_element_type=jnp.float32)
    h = jnp.maximum(h + b0_ref[...], 0.0)
    h0_ref[...] = h
    dis_ref[...] = dis
    hs_ref[...] = (h * dis).astype(jnp.bfloat16)


def _tc1(x, W0, b0, degacc):
    return pl.pallas_call(
        _tc1_body,
        grid=(N // BM,),
        in_specs=[
            pl.BlockSpec((BM, IN_C), lambda i: (i, 0)),
            pl.BlockSpec((IN_C, HID), lambda i: (0, 0)),
            pl.BlockSpec((1, HID), lambda i: (0, 0)),
            pl.BlockSpec((NC, BM, 16), lambda i: (0, i, 0)),
        ],
        out_specs=[
            pl.BlockSpec((BM, HID), lambda i: (i, 0)),
            pl.BlockSpec((BM, HID), lambda i: (i, 0)),
            pl.BlockSpec((BM, 1), lambda i: (i, 0)),
        ],
        out_shape=[
            jax.ShapeDtypeStruct((N, HID), jnp.float32),
            jax.ShapeDtypeStruct((N, HID), jnp.bfloat16),
            jax.ShapeDtypeStruct((N, 1), jnp.float32),
        ],
    )(x, W0, b0, degacc)


def _dense_update(bl, seg_ref, h0_ref, dis_ref, w_ref, g_ref, be_ref, rm_ref,
                  rv_ref):
    dis = dis_ref[...]
    seg = seg_ref[...]
    p = (seg[0].astype(jnp.float32) + seg[1].astype(jnp.float32)) * dis
    t = (1.0 - ALPHA) * p + ALPHA * h0_ref[...]
    u = (1.0 - bl) * t + bl * jnp.dot(t, w_ref[...],
                                      preferred_element_type=jnp.float32)
    scale = g_ref[...] * lax.rsqrt(rv_ref[...] + EPS)
    h = jnp.maximum((u - rm_ref[...]) * scale + be_ref[...], 0.0)
    return h, dis


def _tc_layer_body(bl, seg_ref, h0_ref, dis_ref, w_ref, g_ref, be_ref, rm_ref,
                   rv_ref, hsn_ref):
    h, dis = _dense_update(bl, seg_ref, h0_ref, dis_ref, w_ref, g_ref, be_ref,
                           rm_ref, rv_ref)
    hsn_ref[...] = (h * dis).astype(jnp.bfloat16)


_LAYER_SPECS = [
    pl.BlockSpec((NC, BM, HID), lambda i: (0, i, 0)),
    pl.BlockSpec((BM, HID), lambda i: (i, 0)),
    pl.BlockSpec((BM, 1), lambda i: (i, 0)),
    pl.BlockSpec((HID, HID), lambda i: (0, 0)),
    pl.BlockSpec((1, HID), lambda i: (0, 0)),
    pl.BlockSpec((1, HID), lambda i: (0, 0)),
    pl.BlockSpec((1, HID), lambda i: (0, 0)),
    pl.BlockSpec((1, HID), lambda i: (0, 0)),
]


def _tc_layer(bl, seg, h0, dis, W, g, be, rm, rv):
    return pl.pallas_call(
        functools.partial(_tc_layer_body, bl),
        grid=(N // BM,),
        in_specs=_LAYER_SPECS,
        out_specs=pl.BlockSpec((BM, HID), lambda i: (i, 0)),
        out_shape=jax.ShapeDtypeStruct((N, HID), jnp.bfloat16),
    )(seg, h0, dis, W, g, be, rm, rv)


def _tc_last_body(bl, seg_ref, h0_ref, dis_ref, w_ref, g_ref, be_ref, rm_ref,
                  rv_ref, w1_ref, b1_ref, out_ref):
    h, _ = _dense_update(bl, seg_ref, h0_ref, dis_ref, w_ref, g_ref, be_ref,
                         rm_ref, rv_ref)
    o = jnp.dot(h, w1_ref[...], preferred_element_type=jnp.float32)
    o = o + b1_ref[...]
    z = o - jnp.max(o, axis=1, keepdims=True)
    out_ref[...] = z - jnp.log(jnp.sum(jnp.exp(z), axis=1, keepdims=True))


def _tc_last(bl, seg, h0, dis, W, g, be, rm, rv, W1, b1):
    return pl.pallas_call(
        functools.partial(_tc_last_body, bl),
        grid=(N // BM,),
        in_specs=_LAYER_SPECS + [
            pl.BlockSpec((HID, OUT_C), lambda i: (0, 0)),
            pl.BlockSpec((1, OUT_C), lambda i: (0, 0)),
        ],
        out_specs=pl.BlockSpec((BM, OUT_C), lambda i: (i, 0)),
        out_shape=jax.ShapeDtypeStruct((N, OUT_C), jnp.float32),
    )(seg, h0, dis, W, g, be, rm, rv, W1, b1)


def kernel(x, edge_index, W0, b0, Ws, gammas, betas, rmeans, rvars, W1, b1):
    row = edge_index[0]
    col = edge_index[1]

    rpad = jnp.pad(row.reshape(NC * NS, EPT), ((0, 0), (0, KCH * CH - EPT)))
    cpad = jnp.pad(col.reshape(NC * NS, EPT), ((0, 0), (0, KCH * CH - EPT)),
                   constant_values=N)
    idx2 = jnp.stack([rpad.reshape(NC * NS, KCH, CH),
                      cpad.reshape(NC * NS, KCH, CH)],
                     axis=2).reshape(NC * NS * KCH, 2, CH)
    zeros16 = jnp.zeros((N + PAD, 16), jnp.float32)
    e0 = jnp.zeros((CH, 16), jnp.float32).at[:, 0].set(1.0)
    zbf = jnp.zeros((N, HID), jnp.bfloat16)

    degacc = _sc_degree(cpad.reshape(NC * NS, KCH, CH), zeros16, e0)
    h0, hs, dis = _tc1(x, W0, b0.reshape(1, HID), degacc)

    out = None
    for layer in range(L):
        seg = _sc_prop(hs, zbf, idx2)
        bl = float(np.log(THETA / (layer + 1) + 1.0))
        g = gammas[layer].reshape(1, HID)
        be = betas[layer].reshape(1, HID)
        rm = rmeans[layer].reshape(1, HID)
        rv = rvars[layer].reshape(1, HID)
        if layer < L - 1:
            hs = _tc_layer(bl, seg, h0, dis, Ws[layer], g, be, rm, rv)
        else:
            out = _tc_last(bl, seg, h0, dis, Ws[layer], g, be, rm, rv, W1,
                           b1.reshape(1, OUT_C))
    return out


# speedup vs baseline: 1.4633x; 1.0893x over previous
import functools

import numpy as np
import jax
import jax.numpy as jnp
from jax import lax
from jax.experimental import pallas as pl
from jax.experimental.pallas import tpu as pltpu
from jax.experimental.pallas import tpu_sc as plsc

N = 10000
E = 320000
IN_C = 128
HID = 256
OUT_C = 64
L = 4
ALPHA = 0.1
THETA = 0.5
EPS = 1e-5

NC = 2
NS = 16
CH = 128
PAD = 16
A_OFF = 624
A_SZ = 640

EPT = E // (NC * NS)
KCH = -(-EPT // CH)
KCH_DEG = KCH

_MESH = plsc.VectorSubcoreMesh(core_axis_name="c", subcore_axis_name="s")
_SC_PARAMS = pltpu.CompilerParams(use_tc_tiling_on_sc=False)


def _sc_degree_body(colpad, zeros16, e0, degacc_out, col_v, e0_v, acc_sh, sem):
    c = lax.axis_index("c")
    s = lax.axis_index("s")
    w = c * NS + s
    pltpu.sync_copy(zeros16.at[pl.ds(s * A_OFF, A_SZ)],
                    acc_sh.at[pl.ds(s * A_OFF, A_SZ)])
    pltpu.sync_copy(e0, e0_v)
    pltpu.sync_copy(colpad.at[w], col_v)
    plsc.subcore_barrier()

    def body(k, carry):
        pltpu.sync_copy(e0_v, acc_sh.at[col_v.at[k]], add=True)
        return carry

    lax.fori_loop(0, KCH_DEG, body, 0)
    plsc.subcore_barrier()
    pltpu.sync_copy(acc_sh.at[pl.ds(s * A_OFF, A_SZ)],
                    degacc_out.at[c, pl.ds(s * A_OFF, A_SZ)])


_sc_degree = pl.kernel(
    _sc_degree_body,
    out_type=jax.ShapeDtypeStruct((NC, N, 16), jnp.float32),
    mesh=_MESH,
    scratch_types=[
        pltpu.VMEM((KCH_DEG, CH), jnp.int32),
        pltpu.VMEM((CH, 16), jnp.float32),
        pltpu.VMEM_SHARED((N + PAD, 16), jnp.float32),
        pltpu.SemaphoreType.DMA,
    ],
    compiler_params=_SC_PARAMS,
)


def _sc_prop_body(hs, z, idx2, seg_out, idx_v, rows_v, acc_sh, sem):
    c = lax.axis_index("c")
    s = lax.axis_index("s")
    w = c * NS + s

    pltpu.sync_copy(idx2.at[pl.ds(w * KCH, KCH)], idx_v)

    @pl.when(c == 0)
    def _():
        pltpu.sync_copy(hs.at[pl.ds(s * A_OFF, A_SZ)],
                        acc_sh.at[pl.ds(s * A_OFF, A_SZ)])

    @pl.when(c != 0)
    def _():
        pltpu.sync_copy(z.at[pl.ds(s * A_OFF, A_SZ)],
                        acc_sh.at[pl.ds(s * A_OFF, A_SZ)])

    plsc.subcore_barrier()

    def body(k, carry):
        pltpu.async_copy(hs.at[idx_v.at[k, 0]], rows_v, sem).wait()
        pltpu.sync_copy(rows_v, acc_sh.at[idx_v.at[k, 1]], add=True)
        return carry

    lax.fori_loop(0, KCH, body, 0)
    plsc.subcore_barrier()
    pltpu.sync_copy(acc_sh.at[pl.ds(s * A_OFF, A_SZ)],
                    seg_out.at[c, pl.ds(s * A_OFF, A_SZ)])


_sc_prop = pl.kernel(
    _sc_prop_body,
    out_type=jax.ShapeDtypeStruct((NC, N, HID), jnp.bfloat16),
    mesh=_MESH,
    scratch_types=[
        pltpu.VMEM((KCH, 2, CH), jnp.int32),
        pltpu.VMEM((CH, HID), jnp.bfloat16),
        pltpu.VMEM_SHARED((N + PAD, HID), jnp.bfloat16),
        pltpu.SemaphoreType.DMA,
    ],
    compiler_params=_SC_PARAMS,
)


BM = 2000


def _tc1_body(x_ref, w0_ref, b0_ref, da_ref, h0_ref, hs_ref, dis_ref):
    da = da_ref[...]
    deg = da[0][:, 0:1] + da[1][:, 0:1] + 1.0
    dis = lax.rsqrt(deg)
    h = jnp.dot(x_ref[...], w0_ref[...], preferred_element_type=jnp.float32)
    h = jnp.maximum(h + b0_ref[...], 0.0)
    h0_ref[...] = h
    dis_ref[...] = dis
    hs_ref[...] = (h * dis).astype(jnp.bfloat16)


def _tc1(x, W0, b0, degacc):
    return pl.pallas_call(
        _tc1_body,
        grid=(N // BM,),
        in_specs=[
            pl.BlockSpec((BM, IN_C), lambda i: (i, 0)),
            pl.BlockSpec((IN_C, HID), lambda i: (0, 0)),
            pl.BlockSpec((1, HID), lambda i: (0, 0)),
            pl.BlockSpec((NC, BM, 16), lambda i: (0, i, 0)),
        ],
        out_specs=[
            pl.BlockSpec((BM, HID), lambda i: (i, 0)),
            pl.BlockSpec((BM, HID), lambda i: (i, 0)),
            pl.BlockSpec((BM, 1), lambda i: (i, 0)),
        ],
        out_shape=[
            jax.ShapeDtypeStruct((N, HID), jnp.float32),
            jax.ShapeDtypeStruct((N, HID), jnp.bfloat16),
            jax.ShapeDtypeStruct((N, 1), jnp.float32),
        ],
    )(x, W0, b0, degacc)


def _dense_update(bl, seg_ref, h0_ref, dis_ref, w_ref, g_ref, be_ref, rm_ref,
                  rv_ref):
    dis = dis_ref[...]
    seg = seg_ref[...]
    p = (seg[0].astype(jnp.float32) + seg[1].astype(jnp.float32)) * dis
    t = (1.0 - ALPHA) * p + ALPHA * h0_ref[...]
    u = (1.0 - bl) * t + bl * jnp.dot(t, w_ref[...],
                                      preferred_element_type=jnp.float32)
    scale = g_ref[...] * lax.rsqrt(rv_ref[...] + EPS)
    h = jnp.maximum((u - rm_ref[...]) * scale + be_ref[...], 0.0)
    return h, dis


def _tc_layer_body(bl, seg_ref, h0_ref, dis_ref, w_ref, g_ref, be_ref, rm_ref,
                   rv_ref, hsn_ref):
    h, dis = _dense_update(bl, seg_ref, h0_ref, dis_ref, w_ref, g_ref, be_ref,
                           rm_ref, rv_ref)
    hsn_ref[...] = (h * dis).astype(jnp.bfloat16)


_LAYER_SPECS = [
    pl.BlockSpec((NC, BM, HID), lambda i: (0, i, 0)),
    pl.BlockSpec((BM, HID), lambda i: (i, 0)),
    pl.BlockSpec((BM, 1), lambda i: (i, 0)),
    pl.BlockSpec((HID, HID), lambda i: (0, 0)),
    pl.BlockSpec((1, HID), lambda i: (0, 0)),
    pl.BlockSpec((1, HID), lambda i: (0, 0)),
    pl.BlockSpec((1, HID), lambda i: (0, 0)),
    pl.BlockSpec((1, HID), lambda i: (0, 0)),
]


def _tc_layer(bl, seg, h0, dis, W, g, be, rm, rv):
    return pl.pallas_call(
        functools.partial(_tc_layer_body, bl),
        grid=(N // BM,),
        in_specs=_LAYER_SPECS,
        out_specs=pl.BlockSpec((BM, HID), lambda i: (i, 0)),
        out_shape=jax.ShapeDtypeStruct((N, HID), jnp.bfloat16),
    )(seg, h0, dis, W, g, be, rm, rv)


def _tc_last_body(bl, seg_ref, h0_ref, dis_ref, w_ref, g_ref, be_ref, rm_ref,
                  rv_ref, w1_ref, b1_ref, out_ref):
    h, _ = _dense_update(bl, seg_ref, h0_ref, dis_ref, w_ref, g_ref, be_ref,
                         rm_ref, rv_ref)
    o = jnp.dot(h, w1_ref[...], preferred_element_type=jnp.float32)
    o = o + b1_ref[...]
    z = o - jnp.max(o, axis=1, keepdims=True)
    out_ref[...] = z - jnp.log(jnp.sum(jnp.exp(z), axis=1, keepdims=True))


def _tc_last(bl, seg, h0, dis, W, g, be, rm, rv, W1, b1):
    return pl.pallas_call(
        functools.partial(_tc_last_body, bl),
        grid=(N // BM,),
        in_specs=_LAYER_SPECS + [
            pl.BlockSpec((HID, OUT_C), lambda i: (0, 0)),
            pl.BlockSpec((1, OUT_C), lambda i: (0, 0)),
        ],
        out_specs=pl.BlockSpec((BM, OUT_C), lambda i: (i, 0)),
        out_shape=jax.ShapeDtypeStruct((N, OUT_C), jnp.float32),
    )(seg, h0, dis, W, g, be, rm, rv, W1, b1)


def kernel(x, edge_index, W0, b0, Ws, gammas, betas, rmeans, rvars, W1, b1):
    row = edge_index[0]
    col = edge_index[1]

    rpad = jnp.pad(row.reshape(NC * NS, EPT), ((0, 0), (0, KCH * CH - EPT)))
    cpad = jnp.pad(col.reshape(NC * NS, EPT), ((0, 0), (0, KCH * CH - EPT)),
                   constant_values=N)
    idx2 = jnp.stack([rpad.reshape(NC * NS, KCH, CH),
                      cpad.reshape(NC * NS, KCH, CH)],
                     axis=2).reshape(NC * NS * KCH, 2, CH)
    zeros16 = jnp.zeros((N + PAD, 16), jnp.float32)
    e0 = jnp.zeros((CH, 16), jnp.float32).at[:, 0].set(1.0)
    zbf = jnp.zeros((N, HID), jnp.bfloat16)

    degacc = _sc_degree(cpad.reshape(NC * NS, KCH, CH), zeros16, e0)
    h0, hs, dis = _tc1(x, W0, b0.reshape(1, HID), degacc)

    out = None
    for layer in range(L):
        seg = _sc_prop(hs, zbf, idx2)
        bl = float(np.log(THETA / (layer + 1) + 1.0))
        g = gammas[layer].reshape(1, HID)
        be = betas[layer].reshape(1, HID)
        rm = rmeans[layer].reshape(1, HID)
        rv = rvars[layer].reshape(1, HID)
        if layer < L - 1:
            hs = _tc_layer(bl, seg, h0, dis, Ws[layer], g, be, rm, rv)
        else:
            out = _tc_last(bl, seg, h0, dis, Ws[layer], g, be, rm, rv, W1,
                           b1.reshape(1, OUT_C))
    return out
